# baseline (speedup 1.0000x reference)
import functools
import math
import jax
import jax.numpy as jnp
from jax import lax
from jax.experimental import pallas as pl
from jax.experimental.pallas import tpu as pltpu

_OFFSET = 2
_PAD = 1


def _pos_body(inp_ref, pos_ref):
    x = inp_ref[...]
    mask = (x != _PAD).astype(jnp.int32)
    c = mask
    k = 1
    n = x.shape[1]
    zrow = jnp.zeros_like(c)
    while k < n:
        shifted = jnp.concatenate([zrow[:, :k], c[:, :-k]], axis=1)
        c = c + shifted
        k *= 2
    pos_ref[...] = c * mask + (_OFFSET - 1)


def _positions(inp):
    return pl.pallas_call(
        _pos_body,
        out_shape=jax.ShapeDtypeStruct(inp.shape, jnp.int32),
    )(inp)


def _sincos_body(pos_ref, out_ref, *, half):
    p = pos_ref[0, 0, :].astype(jnp.float32)
    scale = math.log(10000.0) / (half - 1)
    j = lax.broadcasted_iota(jnp.int32, (1, half), 1).astype(jnp.float32)
    f = jnp.exp(j * (-scale))
    arg = p[:, None] * f
    m = (p[:, None] != jnp.float32(_OFFSET - 1)).astype(jnp.float32)
    out_ref[0, 0, :, :half] = jnp.sin(arg) * m
    out_ref[0, 0, :, half:] = jnp.cos(arg) * m


@functools.lru_cache(maxsize=None)
def _make_sincos(B, S, D, BS):
    half = D // 2
    G = (B * S) // BS
    return pl.pallas_call(
        functools.partial(_sincos_body, half=half),
        grid=(G,),
        in_specs=[pl.BlockSpec((1, 1, BS), lambda i: (i, 0, 0))],
        out_specs=pl.BlockSpec((1, 1, BS, D), lambda i: (i, 0, 0, 0)),
        out_shape=jax.ShapeDtypeStruct((G, 1, BS, D), jnp.float32),
    )


def kernel(input, weights):
    bsz, seq_len = input.shape
    D = weights.shape[1]
    BS = 512
    positions = _positions(input).reshape((bsz * seq_len) // BS, 1, BS)
    out = _make_sincos(bsz, seq_len, D, BS)(positions)
    return out.reshape(bsz, seq_len, D)
